# baseline (speedup 1.0000x reference)
import functools

import jax
import jax.numpy as jnp
from jax import lax
from jax.experimental import pallas as pl
from jax.experimental.pallas import tpu as pltpu
from jax.experimental.pallas import tpu_sc as plsc

_SCALE = 8.0


def _make_gather(V2, N, b_per_w, chunk):
    n_chunks = b_per_w // chunk
    mesh = plsc.VectorSubcoreMesh(core_axis_name="c", subcore_axis_name="s")

    @functools.partial(
        pl.kernel,
        mesh=mesh,
        out_type=jax.ShapeDtypeStruct((N, 64), jnp.float32),
        scratch_types=[
            pltpu.VMEM((chunk,), jnp.int32),
            pltpu.VMEM((chunk,), jnp.int32),
            pltpu.VMEM((chunk, 128), jnp.float32),
            pltpu.VMEM((chunk, 64), jnp.float32),
            pltpu.SemaphoreType.DMA,
        ],
        compiler_params=pltpu.CompilerParams(
            use_tc_tiling_on_sc=True, needs_layout_passes=False
        ),
    )
    def gather_kernel(table_hbm, idx2_hbm, sel_hbm, out_hbm,
                      idx_v, sel_v, rows2_v, rows_v, sem):
        wid = lax.axis_index("s") * 2 + lax.axis_index("c")
        wbase = wid * b_per_w

        def chunk_body(g, carry):
            base = wbase + g * chunk
            pltpu.sync_copy(idx2_hbm.at[pl.ds(base, chunk)], idx_v)
            pltpu.sync_copy(sel_hbm.at[pl.ds(base, chunk)], sel_v)
            pltpu.async_copy(table_hbm.at[idx_v], rows2_v, sem).wait()

            def select_row(r, c2):
                off16 = plsc.load_gather(sel_v, [jnp.full((16,), r, jnp.int32)])
                mask = off16 > 0
                for c in range(4):
                    a = rows2_v[r, pl.ds(c * 16, 16)]
                    b = rows2_v[r, pl.ds(64 + c * 16, 16)]
                    rows_v[r, pl.ds(c * 16, 16)] = (
                        jnp.where(mask, b, a) * _SCALE
                    )
                return c2

            lax.fori_loop(0, chunk, select_row, 0)
            pltpu.sync_copy(rows_v, out_hbm.at[pl.ds(base, chunk)])
            return carry

        lax.fori_loop(0, n_chunks, chunk_body, 0)

    return gather_kernel


def kernel(x, input_embedding_table):
    B, L = x.shape
    V, D = input_embedding_table.shape
    N = B * L
    NW = 32
    b_per_w = N // NW
    chunk = 256
    idx = x.reshape(N)
    idx2 = idx >> 1
    sel = idx & 1
    table2 = input_embedding_table.reshape(V // 2, 2 * D)
    out = _make_gather(V // 2, N, b_per_w, chunk)(table2, idx2, sel)
    return out.reshape(B, L, D)


# speedup vs baseline: 1.5179x; 1.5179x over previous
import functools

import jax
import jax.numpy as jnp
from jax import lax
from jax.experimental import pallas as pl
from jax.experimental.pallas import tpu as pltpu
from jax.experimental.pallas import tpu_sc as plsc

_SCALE = 8.0


def _make_gather(V, D, N, b_per_w, chunk):
    n_pairs = b_per_w // (2 * chunk)
    mesh = plsc.VectorSubcoreMesh(core_axis_name="c", subcore_axis_name="s")

    @functools.partial(
        pl.kernel,
        mesh=mesh,
        out_type=jax.ShapeDtypeStruct((N, D), jnp.float32),
        scratch_types=[
            pltpu.VMEM((b_per_w,), jnp.int32),
            pltpu.VMEM((chunk, D), jnp.float32),
            pltpu.VMEM((chunk, D), jnp.float32),
            pltpu.SemaphoreType.DMA,
            pltpu.SemaphoreType.DMA,
            pltpu.SemaphoreType.DMA,
            pltpu.SemaphoreType.DMA,
        ],
        compiler_params=pltpu.CompilerParams(use_tc_tiling_on_sc=False),
    )
    def gather_kernel(table_hbm, idx_hbm, out_hbm,
                      idx_v, rows0_v, rows1_v, gs0, gs1, ws0, ws1):
        wid = lax.axis_index("s") * 2 + lax.axis_index("c")
        wbase = wid * b_per_w
        rows = (rows0_v, rows1_v)
        gsem = (gs0, gs1)
        wsem = (ws0, ws1)

        pltpu.sync_copy(idx_hbm.at[pl.ds(wbase, b_per_w)], idx_v)

        def fire_gather(g, b):
            pltpu.async_copy(
                table_hbm.at[idx_v.at[pl.ds(g * chunk, chunk)]],
                rows[b], gsem[b],
            )

        def wait_gather(b):
            pltpu.make_async_copy(
                table_hbm.at[idx_v.at[pl.ds(0, chunk)]], rows[b], gsem[b]
            ).wait()

        def fire_writeback(g, b):
            pltpu.async_copy(
                rows[b], out_hbm.at[pl.ds(wbase + g * chunk, chunk)], wsem[b]
            )

        def wait_writeback(b):
            pltpu.make_async_copy(
                rows[b], out_hbm.at[pl.ds(wbase, chunk)], wsem[b]
            ).wait()

        def scale(b):
            buf = rows[b]

            def scale8(r8, c2):
                r0 = r8 * 8
                for u in range(8):
                    for cc in range(D // 16):
                        buf[r0 + u, pl.ds(cc * 16, 16)] = (
                            buf[r0 + u, pl.ds(cc * 16, 16)] * _SCALE
                        )
                return c2

            lax.fori_loop(0, chunk // 8, scale8, 0)

        fire_gather(0, 0)

        def pair_body(k, carry):
            g0 = 2 * k

            @pl.when(k > 0)
            def _():
                wait_writeback(1)

            fire_gather(g0 + 1, 1)
            wait_gather(0)
            scale(0)
            fire_writeback(g0, 0)

            @pl.when(k < n_pairs - 1)
            def _():
                wait_writeback(0)
                fire_gather(g0 + 2, 0)

            wait_gather(1)
            scale(1)
            fire_writeback(g0 + 1, 1)
            return carry

        lax.fori_loop(0, n_pairs, pair_body, 0)
        wait_writeback(0)
        wait_writeback(1)

    return gather_kernel


def kernel(x, input_embedding_table):
    B, L = x.shape
    V, D = input_embedding_table.shape
    N = B * L
    NW = 32
    b_per_w = N // NW
    chunk = 640
    idx = x.reshape(N)
    out = _make_gather(V, D, N, b_per_w, chunk)(input_embedding_table, idx)
    return out.reshape(B, L, D)
